# Initial kernel scaffold; baseline (speedup 1.0000x reference)
import jax
import jax.numpy as jnp
from jax.experimental import pallas as pl


def kernel(x, embedding):
    raise NotImplementedError("write your pallas kernel here")


# speedup vs baseline: 17.6443x; 17.6443x over previous
import functools

import jax
import jax.numpy as jnp
from jax.experimental import pallas as pl
from jax.experimental.pallas import tpu as pltpu

_WIN = 5
_PAD = (_WIN - 1) // 2


def _vq_loss_body(x_ref, e_ref, out_ref, *, Tlen, tblk):
    xb = x_ref[0]
    e = e_ref[...]
    e2 = jnp.sum(e * e, axis=0)
    ebf = e.astype(jnp.bfloat16)

    f2 = jnp.float32(0.0)
    for w in range(_WIN):
        s = xb[:, w:w + Tlen]
        f2 = f2 + jnp.sum(s * s)

    ft = jnp.concatenate(
        [xb[:, w:w + Tlen] for w in range(_WIN)], axis=0
    ).astype(jnp.bfloat16)

    acc = jnp.float32(0.0)
    for t0 in range(0, Tlen, tblk):
        g = jax.lax.dot_general(
            ft[:, t0:t0 + tblk], ebf,
            dimension_numbers=(((0,), (0,)), ((), ())),
            preferred_element_type=jnp.float32,
        )
        m = jnp.min(e2[None, :] - 2.0 * g, axis=1)
        acc = acc + jnp.sum(m)

    out_ref[0, 0, 0] = acc + f2


@jax.jit
def kernel(x, embedding):
    B, P, T = x.shape
    K, D = embedding.shape
    xp = jnp.pad(x, ((0, 0), (0, 0), (_PAD, _PAD)))
    et = jnp.transpose(embedding.reshape(K, P, _WIN), (2, 1, 0)).reshape(D, K)

    body = functools.partial(_vq_loss_body, Tlen=T, tblk=2048)
    partials = pl.pallas_call(
        body,
        grid=(B,),
        in_specs=[
            pl.BlockSpec((1, P, T + 2 * _PAD), lambda b: (b, 0, 0)),
            pl.BlockSpec((D, K), lambda b: (0, 0)),
        ],
        out_specs=pl.BlockSpec((1, 1, 1), lambda b: (b, 0, 0), memory_space=pltpu.SMEM),
        out_shape=jax.ShapeDtypeStruct((B, 1, 1), jnp.float32),
    )(xp, et)
    total = jnp.sum(partials)
    return 0.25 * total / (B * T * D)


# speedup vs baseline: 19.9163x; 1.1288x over previous
import functools

import jax
import jax.numpy as jnp
from jax.experimental import pallas as pl
from jax.experimental.pallas import tpu as pltpu

_WIN = 5
_PAD = (_WIN - 1) // 2


def _vq_loss_body(x_ref, e_ref, out_ref, *, Tlen, tblk):
    xb = x_ref[0]
    e = e_ref[...]
    e2 = jnp.sum(e * e, axis=0)
    ebf = jnp.concatenate(
        [e, (-0.5 * e2)[None, :]], axis=0
    ).astype(jnp.bfloat16)

    f2 = jnp.float32(0.0)
    for w in range(_WIN):
        s = xb[:, w:w + Tlen]
        f2 = f2 + jnp.sum(s * s)

    xbb = xb.astype(jnp.bfloat16)
    ft = jnp.concatenate(
        [xbb[:, w:w + Tlen] for w in range(_WIN)]
        + [jnp.ones((1, Tlen), jnp.bfloat16)],
        axis=0,
    )

    acc = jnp.float32(0.0)
    for t0 in range(0, Tlen, tblk):
        g = jax.lax.dot_general(
            ft[:, t0:t0 + tblk], ebf,
            dimension_numbers=(((0,), (0,)), ((), ())),
            preferred_element_type=jnp.float32,
        )
        acc = acc + jnp.sum(jnp.max(g, axis=1))

    out_ref[0, 0, 0] = f2 - 2.0 * acc


@jax.jit
def kernel(x, embedding):
    B, P, T = x.shape
    K, D = embedding.shape
    xp = jnp.pad(x, ((0, 0), (0, 0), (_PAD, _PAD)))
    et = jnp.transpose(embedding.reshape(K, P, _WIN), (2, 1, 0)).reshape(D, K)

    body = functools.partial(_vq_loss_body, Tlen=T, tblk=2048)
    partials = pl.pallas_call(
        body,
        grid=(B,),
        in_specs=[
            pl.BlockSpec((1, P, T + 2 * _PAD), lambda b: (b, 0, 0)),
            pl.BlockSpec((D, K), lambda b: (0, 0)),
        ],
        out_specs=pl.BlockSpec((1, 1, 1), lambda b: (b, 0, 0), memory_space=pltpu.SMEM),
        out_shape=jax.ShapeDtypeStruct((B, 1, 1), jnp.float32),
    )(xp, et)
    total = jnp.sum(partials)
    return 0.25 * total / (B * T * D)


# speedup vs baseline: 26.9627x; 1.3538x over previous
import functools

import jax
import jax.numpy as jnp
from jax.experimental import pallas as pl
from jax.experimental.pallas import tpu as pltpu

_WIN = 5
_PAD = (_WIN - 1) // 2


def _vq_loss_body(x_ref, e_ref, out_ref, *, Tlen, tblk):
    xb = x_ref[0]
    e = e_ref[...]
    e2 = jnp.sum(e * e, axis=0)
    ebf = jnp.concatenate(
        [e, (-0.5 * e2)[None, :]], axis=0
    ).astype(jnp.bfloat16)

    s_all = jnp.sum(xb * xb)
    c0 = xb[:, _PAD:_PAD + 1]
    c1 = xb[:, _PAD + 1:_PAD + 2]
    c2 = xb[:, Tlen:Tlen + 1]
    c3 = xb[:, Tlen + 1:Tlen + 2]
    corr = (2.0 * jnp.sum(c0 * c0) + jnp.sum(c1 * c1)
            + jnp.sum(c2 * c2) + 2.0 * jnp.sum(c3 * c3))
    f2 = 5.0 * s_all - corr

    xbb = xb.astype(jnp.bfloat16)
    ft = jnp.concatenate(
        [xbb[:, w:w + Tlen] for w in range(_WIN)]
        + [jnp.ones((1, Tlen), jnp.bfloat16)],
        axis=0,
    )

    acc_v = jnp.zeros((tblk,), jnp.float32)
    for t0 in range(0, Tlen, tblk):
        g = jax.lax.dot_general(
            ebf, ft[:, t0:t0 + tblk],
            dimension_numbers=(((0,), (0,)), ((), ())),
            preferred_element_type=jnp.float32,
        )
        acc_v = acc_v + jnp.max(g, axis=0)

    out_ref[0, 0, 0] = f2 - 2.0 * jnp.sum(acc_v)


@jax.jit
def kernel(x, embedding):
    B, P, T = x.shape
    K, D = embedding.shape
    xp = jnp.pad(x, ((0, 0), (0, 0), (_PAD, _PAD)))
    et = jnp.transpose(embedding.reshape(K, P, _WIN), (2, 1, 0)).reshape(D, K)

    body = functools.partial(_vq_loss_body, Tlen=T, tblk=2048)
    partials = pl.pallas_call(
        body,
        grid=(B,),
        in_specs=[
            pl.BlockSpec((1, P, T + 2 * _PAD), lambda b: (b, 0, 0)),
            pl.BlockSpec((D, K), lambda b: (0, 0)),
        ],
        out_specs=pl.BlockSpec((1, 1, 1), lambda b: (b, 0, 0), memory_space=pltpu.SMEM),
        out_shape=jax.ShapeDtypeStruct((B, 1, 1), jnp.float32),
    )(xp, et)
    total = jnp.sum(partials)
    return 0.25 * total / (B * T * D)
